# Initial kernel scaffold; baseline (speedup 1.0000x reference)
import jax
import jax.numpy as jnp
from jax.experimental import pallas as pl


def kernel(feature_a, feature_b, matrix, Wmlp, bmlp, att, W1, b1, alpha, W2, b2, edges):
    raise NotImplementedError("write your pallas kernel here")


# speedup vs baseline: 42.5617x; 42.5617x over previous
import functools

import jax
import jax.numpy as jnp
from jax import lax
from jax.experimental import pallas as pl


_N = 64
_D = 128
_E = 128


def _fused_kernel(src_ref, dst_ref, fa_ref, fb_ref, mat_ref, wmlp_ref,
                  bmlp_ref, att_ref, w1_ref, b1_ref, alpha_ref, w2_ref,
                  b2_ref, out_a_ref, out_b_ref):
    fa = fa_ref[...]
    fb = fb_ref[...]
    mat = mat_ref[...]
    alpha = alpha_ref[0, 0]

    iota_en = lax.broadcasted_iota(jnp.int32, (_E, _N), 1)
    iota_ne = lax.broadcasted_iota(jnp.int32, (_N, _E), 0)

    def agg(i, f_src, f_msg, sign_mode):
        new_emb = jnp.dot(f_msg, wmlp_ref[i], preferred_element_type=jnp.float32)
        new_emb = new_emb + bmlp_ref[i][None, :]

        src = src_ref[i]
        dst = dst_ref[i]
        s_en = (src[:, None] == iota_en).astype(jnp.float32)
        d_en = (dst[:, None] == iota_en).astype(jnp.float32)
        s_ne = (src[None, :] == iota_ne).astype(jnp.float32)

        h_a = jnp.dot(s_en, f_src, preferred_element_type=jnp.float32)
        h_b = jnp.dot(d_en, new_emb, preferred_element_type=jnp.float32)

        att_i = att_ref[i]
        logits = jnp.dot(h_a, att_i[:_D, :], preferred_element_type=jnp.float32)
        logits_b = jnp.dot(h_b, att_i[_D:, :], preferred_element_type=jnp.float32)
        if sign_mode == 1:
            w_e = jnp.sum(jnp.dot(s_en, mat, preferred_element_type=jnp.float32)
                          * d_en, axis=1, keepdims=True)
            logits = logits + w_e * logits_b
        elif sign_mode == 2:
            w_e = jnp.sum(jnp.dot(d_en, mat, preferred_element_type=jnp.float32)
                          * s_en, axis=1, keepdims=True)
            logits = logits + w_e * logits_b
        else:
            logits = logits + logits_b

        elu = jnp.where(logits >= 0, logits, 0.1 * (jnp.exp(logits) - 1.0))
        vals = jnp.exp(elu)

        row_sum = jnp.dot(s_ne, vals, preferred_element_type=jnp.float32)
        out_sum = jnp.dot(s_ne, vals * h_b, preferred_element_type=jnp.float32)
        row_sum = jnp.where(row_sum == 0.0, 1.0, row_sum)
        return out_sum / row_sum

    def update(x):
        h = jnp.dot(x, w1_ref[...], preferred_element_type=jnp.float32)
        h = h + b1_ref[...][None, :]
        h = jnp.where(h >= 0, h, alpha * h)
        h = jnp.dot(h, w2_ref[...], preferred_element_type=jnp.float32)
        return h + b2_ref[...][None, :]

    m0 = agg(0, fa, fb, 1)
    m1 = agg(1, fa, fb, 0)
    m2 = agg(2, fa, fa, 0)
    m3 = agg(3, fa, fa, 0)
    out_a_ref[...] = update(jnp.concatenate([fa, m0, m1, m2, m3], axis=1))

    m4 = agg(4, fb, fa, 2)
    m5 = agg(5, fb, fa, 0)
    m6 = agg(6, fb, fb, 0)
    m7 = agg(7, fb, fb, 0)
    out_b_ref[...] = update(jnp.concatenate([fb, m4, m5, m6, m7], axis=1))


@jax.jit
def kernel(feature_a, feature_b, matrix, Wmlp, bmlp, att, W1, b1, alpha, W2,
           b2, edges):
    src = edges[:, :, 0].astype(jnp.int32)
    dst = edges[:, :, 1].astype(jnp.int32)
    alpha2d = jnp.reshape(alpha.astype(jnp.float32), (1, 1))
    out_a, out_b = pl.pallas_call(
        _fused_kernel,
        out_shape=(
            jax.ShapeDtypeStruct((_N, _D), jnp.float32),
            jax.ShapeDtypeStruct((_N, _D), jnp.float32),
        ),
    )(src, dst, feature_a, feature_b, matrix, Wmlp, bmlp, att, W1, b1,
      alpha2d, W2, b2)
    return (out_a, out_b)
